# baseline (speedup 1.0000x reference)
import jax
import jax.numpy as jnp
from jax.experimental import pallas as pl
from jax.experimental.pallas import tpu as pltpu

N_TOK = 4096
N_CLU = 256
N_FEA = 128


def _body(x_ref, c_ref, out_ref, idx_v, idx_s, sem):
    x = x_ref[:]
    c = c_ref[:]
    xn = jnp.sum(x * x, axis=1, keepdims=True)
    xc = jax.lax.dot_general(
        x, c, (((1,), (1,)), ((), ())),
        preferred_element_type=jnp.float32,
        precision=jax.lax.Precision.HIGHEST,
    )
    scores = xn - 2.0 * xc
    idx_v[0, :] = jnp.argmin(scores, axis=0).astype(jnp.int32)
    copy = pltpu.make_async_copy(idx_v, idx_s, sem)
    copy.start()
    copy.wait()

    def gather_row(k, carry):
        s = idx_s[0, k]
        out_ref[pl.ds(k, 1), :] = x_ref[pl.ds(s, 1), :]
        return carry

    jax.lax.fori_loop(0, N_CLU, gather_row, 0, unroll=16)


def kernel(x, cluster_centers):
    x2 = x.reshape(N_TOK, N_FEA)
    out = pl.pallas_call(
        _body,
        out_shape=jax.ShapeDtypeStruct((N_CLU, N_FEA), jnp.float32),
        scratch_shapes=[
            pltpu.VMEM((1, N_CLU), jnp.int32),
            pltpu.SMEM((1, N_CLU), jnp.int32),
            pltpu.SemaphoreType.DMA,
        ],
    )(x2, cluster_centers)
    return out[None]
